# baseline (speedup 1.0000x reference)
import functools

import jax
import jax.numpy as jnp
from jax import lax
from jax.experimental import pallas as pl
from jax.experimental.pallas import tpu as pltpu
from jax.experimental.pallas import tpu_sc as plsc

_IDX_CHUNK = 128
_TAIL_CHUNK = 448


def kernel(mem, values, idx):
    cap, dim = mem.shape
    nv = values.shape[0]
    info = plsc.get_sparse_core_info()
    nc, ns = info.num_cores, info.num_subcores
    nw = nc * ns
    vpw = nv // nw
    n_idx_chunks = vpw // _IDX_CHUNK
    tail = cap - nv
    tpw = tail // nw
    span = -(-tpw // 8) * 8
    n_full = span // _TAIL_CHUNK
    sizes = [_TAIL_CHUNK] * n_full + ([span - n_full * _TAIL_CHUNK]
                                      if span % _TAIL_CHUNK else [])
    idx3 = idx.astype(jnp.int32).reshape(nw, n_idx_chunks, _IDX_CHUNK)
    mesh = plsc.VectorSubcoreMesh(core_axis_name="c", subcore_axis_name="s")

    @functools.partial(
        pl.kernel,
        out_type=jax.ShapeDtypeStruct((cap, dim), mem.dtype),
        mesh=mesh,
        scratch_types=[
            pltpu.VMEM((n_idx_chunks, _IDX_CHUNK), jnp.int32),
            pltpu.VMEM((vpw, dim), mem.dtype),
            pltpu.VMEM((_TAIL_CHUNK, dim), mem.dtype),
            pltpu.SemaphoreType.DMA,
            pltpu.SemaphoreType.DMA,
        ],
    )
    def sc_kernel(mem_hbm, values_hbm, idx_hbm, out_hbm,
                  idx_v, buf_a, buf_b, sem_a, sem_b):
        wid = lax.axis_index("s") * nc + lax.axis_index("c")

        pltpu.sync_copy(idx_hbm.at[wid], idx_v)
        pltpu.sync_copy(values_hbm.at[pl.ds(wid * vpw, vpw)], buf_a)
        phase_a = [
            pltpu.make_async_copy(
                buf_a.at[pl.ds(j * _IDX_CHUNK, _IDX_CHUNK)],
                out_hbm.at[idx_v.at[j]],
                sem_a,
            )
            for j in range(n_idx_chunks)
        ]
        for cp in phase_a:
            cp.start()

        base = pl.multiple_of(nv + ((wid * tpw) // 8) * 8, 8)
        bufs = (buf_b, buf_a)
        sems = (sem_b, sem_a)
        pending = [None, phase_a]
        off = 0
        for k, n in enumerate(sizes):
            b = k % 2
            if pending[b] is not None:
                for cp in pending[b]:
                    cp.wait()
            pltpu.sync_copy(mem_hbm.at[pl.ds(base + off, n)],
                            bufs[b].at[pl.ds(0, n)])
            cp = pltpu.make_async_copy(bufs[b].at[pl.ds(0, n)],
                                       out_hbm.at[pl.ds(base + off, n)],
                                       sems[b])
            cp.start()
            pending[b] = [cp]
            off += n
        for plist in pending:
            if plist is not None:
                for cp in plist:
                    cp.wait()

    return sc_kernel(mem, values, idx3)
